# baseline (speedup 1.0000x reference)
import functools

import jax
import jax.numpy as jnp
from jax import lax
from jax.experimental import pallas as pl
from jax.experimental.pallas import tpu as pltpu
from jax.experimental.pallas import tpu_sc as plsc

N = 10000
D = 128
E = 320000

NC, NS, L = 2, 16, 16
NW = NC * NS
CH = 80
NCHUNK = E // CH
TILE_CHUNKS = NCHUNK // NW
TILE_EDGES = TILE_CHUNKS * CH

ROWS_MAIN = 624
ROWS_TAIL = N - ROWS_MAIN * NS

ZR = 16


def _fill_rows(ref, nrows, ncols, value):
    per_row = ncols // L

    def body(k, _):
        i = k // per_row
        j = k % per_row
        ref[i, pl.ds(j * L, L)] = jnp.full((L,), value, ref.dtype)
        return 0

    lax.fori_loop(0, nrows * per_row, body, 0)


def _fill_1d(ref, n, value):
    for j in range(n // L):
        ref[pl.ds(j * L, L)] = jnp.full((L,), value, ref.dtype)


def _zero_span(zeros_v, dst, start, nrows, zrows):
    full, rem = nrows // zrows, nrows % zrows
    for k in range(full):
        pltpu.sync_copy(zeros_v, dst.at[pl.ds(start + k * zrows, zrows)])
    if rem:
        pltpu.sync_copy(zeros_v.at[pl.ds(0, rem)],
                        dst.at[pl.ds(start + full * zrows, rem)])


def _zero_tile_rows(zeros_v, dst, sid, zrows):
    _zero_span(zeros_v, dst, sid * ROWS_MAIN, ROWS_MAIN, zrows)

    @pl.when(sid == NS - 1)
    def _():
        _zero_span(zeros_v, dst, ROWS_MAIN * NS, ROWS_TAIL, zrows)


_sc_mesh = plsc.VectorSubcoreMesh(core_axis_name="c", subcore_axis_name="s")


@functools.partial(
    pl.kernel,
    out_type=jax.ShapeDtypeStruct((NC * 2 * N,), jnp.float32),
    mesh=_sc_mesh,
    scratch_types=(
        pltpu.VMEM_SHARED((N,), jnp.float32),
        pltpu.VMEM_SHARED((N,), jnp.float32),
        pltpu.VMEM((TILE_EDGES,), jnp.int32),
        pltpu.VMEM((TILE_EDGES,), jnp.int32),
        pltpu.VMEM((TILE_CHUNKS, CH), jnp.int32),
        pltpu.VMEM((TILE_CHUNKS, CH), jnp.int32),
        pltpu.VMEM((CH,), jnp.float32),
        pltpu.VMEM((ROWS_MAIN,), jnp.float32),
        pltpu.SemaphoreType.DMA,
    ),
)
def _degree_kernel(edge_hbm, cnt_out,
                   cnt_src, cnt_dst, sraw, draw, sidx, didx,
                   ones_v, zeros_v, sem):
    cid = lax.axis_index("c")
    sid = lax.axis_index("s")
    wid = sid * NC + cid

    pltpu.sync_copy(edge_hbm.at[pl.ds(wid * TILE_EDGES, TILE_EDGES)], sraw)
    pltpu.sync_copy(edge_hbm.at[pl.ds(E + wid * TILE_EDGES, TILE_EDGES)], draw)

    def reshape_body(k, _):
        r = k // (CH // L)
        c = k % (CH // L)
        v = k * L
        sidx[r, pl.ds(c * L, L)] = sraw[pl.ds(v, L)]
        didx[r, pl.ds(c * L, L)] = draw[pl.ds(v, L)]
        return 0

    lax.fori_loop(0, TILE_EDGES // L, reshape_body, 0)

    _fill_1d(ones_v, CH, 1.0)
    _fill_1d(zeros_v, ROWS_MAIN, 0.0)
    _zero_tile_rows(zeros_v, cnt_src, sid, ROWS_MAIN)
    _zero_tile_rows(zeros_v, cnt_dst, sid, ROWS_MAIN)
    plsc.subcore_barrier()

    def issue(i, _):
        pltpu.async_copy(ones_v, cnt_src.at[sidx.at[i]], sem, add=True)
        pltpu.async_copy(ones_v, cnt_dst.at[didx.at[i]], sem, add=True)
        return 0

    lax.fori_loop(0, TILE_CHUNKS, issue, 0)

    def drain(i, _):
        pltpu.make_async_copy(ones_v, cnt_src.at[sidx.at[i]], sem).wait()
        pltpu.make_async_copy(ones_v, cnt_dst.at[didx.at[i]], sem).wait()
        return 0

    lax.fori_loop(0, TILE_CHUNKS, drain, 0)

    plsc.subcore_barrier()

    def readout(cnt, out_base):
        r0 = sid * ROWS_MAIN
        pltpu.sync_copy(cnt.at[pl.ds(r0, ROWS_MAIN)], zeros_v)
        pltpu.sync_copy(zeros_v,
                        cnt_out.at[pl.ds(out_base + r0, ROWS_MAIN)])

        @pl.when(sid == NS - 1)
        def _():
            t0 = ROWS_MAIN * NS
            pltpu.sync_copy(cnt.at[pl.ds(t0, ROWS_TAIL)],
                            zeros_v.at[pl.ds(0, ROWS_TAIL)])
            pltpu.sync_copy(zeros_v.at[pl.ds(0, ROWS_TAIL)],
                            cnt_out.at[pl.ds(out_base + t0, ROWS_TAIL)])

    readout(cnt_src, cid * 2 * N)
    readout(cnt_dst, cid * 2 * N + N)


@functools.partial(
    pl.kernel,
    out_type=jax.ShapeDtypeStruct((NC, N, D), jnp.float32),
    mesh=_sc_mesh,
    scratch_types=(
        pltpu.VMEM_SHARED((N, D), jnp.float32),
        pltpu.VMEM((TILE_EDGES,), jnp.int32),
        pltpu.VMEM((TILE_EDGES,), jnp.int32),
        pltpu.VMEM((CH,), jnp.int32),
        pltpu.VMEM((CH,), jnp.int32),
        pltpu.VMEM((CH, D), jnp.float32),
        pltpu.VMEM((CH, D), jnp.float32),
        pltpu.VMEM((ZR, D), jnp.float32),
        pltpu.SemaphoreType.DMA,
        pltpu.SemaphoreType.DMA,
        pltpu.SemaphoreType.DMA,
        pltpu.SemaphoreType.DMA,
    ),
)
def _aggregate_kernel(h_hbm, edge_hbm, part_out,
                      acc, sraw, draw, d0_v, d1_v, r0_v, r1_v, zeros_v,
                      g0, g1, s0, s1):
    cid = lax.axis_index("c")
    sid = lax.axis_index("s")
    wid = sid * NC + cid
    didx = (d0_v, d1_v)
    rows = (r0_v, r1_v)
    gsem = (g0, g1)
    ssem = (s0, s1)

    pltpu.sync_copy(edge_hbm.at[pl.ds(wid * TILE_EDGES, TILE_EDGES)], sraw)
    pltpu.sync_copy(edge_hbm.at[pl.ds(E + wid * TILE_EDGES, TILE_EDGES)], draw)

    def set_didx(j, b):
        for c in range(CH // L):
            didx[b][pl.ds(c * L, L)] = draw[pl.ds(j * CH + c * L, L)]

    def start_gather(j, b):
        pltpu.async_copy(h_hbm.at[sraw.at[pl.ds(j * CH, CH)]],
                         rows[b], gsem[b])

    def wait_gather(b):
        pltpu.make_async_copy(h_hbm.at[sraw.at[pl.ds(0, CH)]],
                              rows[b], gsem[b]).wait()

    def start_scatter(b):
        pltpu.async_copy(rows[b], acc.at[didx[b]], ssem[b], add=True)

    def wait_scatter(b):
        pltpu.make_async_copy(rows[b], acc.at[didx[b]], ssem[b]).wait()

    set_didx(0, 0)
    set_didx(1, 1)
    start_gather(0, 0)
    start_gather(1, 1)

    _fill_rows(zeros_v, ZR, D, 0.0)
    _zero_tile_rows(zeros_v, acc, sid, ZR)
    plsc.subcore_barrier()

    def step(j, b, prefetch):
        wait_gather(b)
        start_scatter(b)
        if prefetch:
            wait_scatter(b)
            set_didx(j + 2, b)
            start_gather(j + 2, b)

    def body(g, _):
        step(2 * g, 0, prefetch=True)
        step(2 * g + 1, 1, prefetch=True)
        return 0

    lax.fori_loop(0, (TILE_CHUNKS - 3) // 2, body, 0)
    step(TILE_CHUNKS - 3, 0, prefetch=True)
    step(TILE_CHUNKS - 2, 1, prefetch=False)
    step(TILE_CHUNKS - 1, 0, prefetch=False)
    wait_scatter(1)
    wait_scatter(0)

    plsc.subcore_barrier()
    r0 = sid * ROWS_MAIN
    pltpu.sync_copy(acc.at[pl.ds(r0, ROWS_MAIN)],
                    part_out.at[cid, pl.ds(r0, ROWS_MAIN)])

    @pl.when(sid == NS - 1)
    def _():
        t0 = ROWS_MAIN * NS
        pltpu.sync_copy(acc.at[pl.ds(t0, ROWS_TAIL)],
                        part_out.at[cid, pl.ds(t0, ROWS_TAIL)])


_BLK = 1000


def _scale_body(node_ref, cnt_ref, h_ref):
    deg = cnt_ref[0] + cnt_ref[1]
    h_ref[...] = node_ref[...] * jax.lax.rsqrt(jnp.maximum(deg, 1.0))


_scale_kernel = pl.pallas_call(
    _scale_body,
    grid=(N // _BLK,),
    in_specs=[
        pl.BlockSpec((_BLK, D), lambda i: (i, 0)),
        pl.BlockSpec((NC, _BLK, 1), lambda i: (0, i, 0)),
    ],
    out_specs=pl.BlockSpec((_BLK, D), lambda i: (i, 0)),
    out_shape=jax.ShapeDtypeStruct((N, D), jnp.float32),
)


def _combine_body(part_ref, cnt_ref, out_ref):
    deg = cnt_ref[0] + cnt_ref[1]
    agg = part_ref[0] + part_ref[1]
    out_ref[...] = agg * jax.lax.rsqrt(jnp.maximum(deg, 1.0))


_combine_kernel = pl.pallas_call(
    _combine_body,
    grid=(N // _BLK,),
    in_specs=[
        pl.BlockSpec((NC, _BLK, D), lambda i: (0, i, 0)),
        pl.BlockSpec((NC, _BLK, 1), lambda i: (0, i, 0)),
    ],
    out_specs=pl.BlockSpec((_BLK, D), lambda i: (i, 0)),
    out_shape=jax.ShapeDtypeStruct((N, D), jnp.float32),
)


def kernel(node_f, edge_index):
    edge_flat = edge_index.astype(jnp.int32).reshape(-1)
    cnt = _degree_kernel(edge_flat).reshape(NC, 2, N)
    cnt_src = cnt[:, 0, :].reshape(NC, N, 1)
    cnt_dst = cnt[:, 1, :].reshape(NC, N, 1)
    h = _scale_kernel(node_f, cnt_src)
    partials = _aggregate_kernel(h, edge_flat)
    return _combine_kernel(partials, cnt_dst)


# speedup vs baseline: 1.1988x; 1.1988x over previous
import functools

import jax
import jax.numpy as jnp
from jax import lax
from jax.experimental import pallas as pl
from jax.experimental.pallas import tpu as pltpu
from jax.experimental.pallas import tpu_sc as plsc

N = 10000
D = 128
E = 320000

NC, NS, L = 2, 16, 16
NW = NC * NS
CH = 128
NCHUNK = E // CH
TILE_CHUNKS = NCHUNK // NW
XTRA = NCHUNK - TILE_CHUNKS * NW
TILE_EDGES = TILE_CHUNKS * CH

NP2 = 10240
CROWS = NP2 // NS

ROWS_MAIN = 624
ROWS_TAIL = N - ROWS_MAIN * NS

ZR = 16


def _fill_rows(ref, nrows, ncols, value):
    per_row = ncols // L

    def body(k, _):
        i = k // per_row
        j = k % per_row
        ref[i, pl.ds(j * L, L)] = jnp.full((L,), value, ref.dtype)
        return 0

    lax.fori_loop(0, nrows * per_row, body, 0)


def _fill_1d(ref, n, value):
    for j in range(n // L):
        ref[pl.ds(j * L, L)] = jnp.full((L,), value, ref.dtype)


def _zero_span(zeros_v, dst, start, nrows, zrows):
    full, rem = nrows // zrows, nrows % zrows
    for k in range(full):
        pltpu.sync_copy(zeros_v, dst.at[pl.ds(start + k * zrows, zrows)])
    if rem:
        pltpu.sync_copy(zeros_v.at[pl.ds(0, rem)],
                        dst.at[pl.ds(start + full * zrows, rem)])


_sc_mesh = plsc.VectorSubcoreMesh(core_axis_name="c", subcore_axis_name="s")


@functools.partial(
    pl.kernel,
    out_type=jax.ShapeDtypeStruct((NC * 2 * NP2,), jnp.float32),
    mesh=_sc_mesh,
    scratch_types=(
        pltpu.VMEM_SHARED((NP2,), jnp.float32),
        pltpu.VMEM_SHARED((NP2,), jnp.float32),
        pltpu.VMEM((TILE_EDGES,), jnp.int32),
        pltpu.VMEM((TILE_EDGES,), jnp.int32),
        pltpu.VMEM((TILE_CHUNKS, CH), jnp.int32),
        pltpu.VMEM((TILE_CHUNKS, CH), jnp.int32),
        pltpu.VMEM((CH,), jnp.int32),
        pltpu.VMEM((CH,), jnp.int32),
        pltpu.VMEM((CH,), jnp.float32),
        pltpu.VMEM((CROWS,), jnp.float32),
        pltpu.SemaphoreType.DMA,
    ),
)
def _degree_kernel(edge_hbm, cnt_out,
                   cnt_src, cnt_dst, sraw, draw, sidx, didx,
                   sidx_x, didx_x, ones_v, zeros_v, sem):
    cid = lax.axis_index("c")
    sid = lax.axis_index("s")
    wid = sid * NC + cid

    pltpu.sync_copy(edge_hbm.at[pl.ds(wid * TILE_EDGES, TILE_EDGES)], sraw)
    pltpu.sync_copy(edge_hbm.at[pl.ds(E + wid * TILE_EDGES, TILE_EDGES)], draw)

    @pl.when(wid < XTRA)
    def _():
        xb = (NW * TILE_CHUNKS + wid) * CH
        pltpu.sync_copy(edge_hbm.at[pl.ds(xb, CH)], sidx_x)
        pltpu.sync_copy(edge_hbm.at[pl.ds(E + xb, CH)], didx_x)

    def reshape_body(k, _):
        r = k // (CH // L)
        c = k % (CH // L)
        v = k * L
        sidx[r, pl.ds(c * L, L)] = sraw[pl.ds(v, L)]
        didx[r, pl.ds(c * L, L)] = draw[pl.ds(v, L)]
        return 0

    lax.fori_loop(0, TILE_EDGES // L, reshape_body, 0)

    _fill_1d(ones_v, CH, 1.0)
    _fill_1d(zeros_v, CROWS, 0.0)
    _zero_span(zeros_v, cnt_src, sid * CROWS, CROWS, CROWS)
    _zero_span(zeros_v, cnt_dst, sid * CROWS, CROWS, CROWS)
    plsc.subcore_barrier()

    def issue(i, _):
        pltpu.async_copy(ones_v, cnt_src.at[sidx.at[i]], sem, add=True)
        pltpu.async_copy(ones_v, cnt_dst.at[didx.at[i]], sem, add=True)
        return 0

    lax.fori_loop(0, TILE_CHUNKS, issue, 0)

    @pl.when(wid < XTRA)
    def _():
        pltpu.async_copy(ones_v, cnt_src.at[sidx_x], sem, add=True)
        pltpu.async_copy(ones_v, cnt_dst.at[didx_x], sem, add=True)

    def drain(i, _):
        pltpu.make_async_copy(ones_v, cnt_src.at[sidx.at[i]], sem).wait()
        pltpu.make_async_copy(ones_v, cnt_dst.at[didx.at[i]], sem).wait()
        return 0

    lax.fori_loop(0, TILE_CHUNKS, drain, 0)

    @pl.when(wid < XTRA)
    def _():
        pltpu.make_async_copy(ones_v, cnt_src.at[sidx_x], sem).wait()
        pltpu.make_async_copy(ones_v, cnt_dst.at[didx_x], sem).wait()

    plsc.subcore_barrier()

    def readout(cnt, out_base):
        r0 = sid * CROWS
        pltpu.sync_copy(cnt.at[pl.ds(r0, CROWS)], zeros_v)
        pltpu.sync_copy(zeros_v, cnt_out.at[pl.ds(out_base + r0, CROWS)])

    readout(cnt_src, cid * 2 * NP2)
    readout(cnt_dst, cid * 2 * NP2 + NP2)


@functools.partial(
    pl.kernel,
    out_type=jax.ShapeDtypeStruct((NC, N, D), jnp.float32),
    mesh=_sc_mesh,
    scratch_types=(
        pltpu.VMEM_SHARED((N, D), jnp.float32),
        pltpu.VMEM((TILE_EDGES,), jnp.int32),
        pltpu.VMEM((CH,), jnp.int32),
        pltpu.VMEM((CH,), jnp.int32),
        pltpu.VMEM((CH,), jnp.int32),
        pltpu.VMEM((CH,), jnp.int32),
        pltpu.VMEM((CH, D), jnp.float32),
        pltpu.VMEM((CH, D), jnp.float32),
        pltpu.VMEM((ZR, D), jnp.float32),
        pltpu.SemaphoreType.DMA,
        pltpu.SemaphoreType.DMA,
        pltpu.SemaphoreType.DMA,
        pltpu.SemaphoreType.DMA,
        pltpu.SemaphoreType.DMA,
        pltpu.SemaphoreType.DMA,
    ),
)
def _aggregate_kernel(h_hbm, edge_hbm, part_out,
                      acc, sraw, d0_v, d1_v, sidx_x, didx_x, r0_v, r1_v,
                      zeros_v, g0, g1, s0, s1, i0, i1):
    cid = lax.axis_index("c")
    sid = lax.axis_index("s")
    wid = sid * NC + cid
    didx = (d0_v, d1_v)
    rows = (r0_v, r1_v)
    gsem = (g0, g1)
    ssem = (s0, s1)
    isem = (i0, i1)

    pltpu.sync_copy(edge_hbm.at[pl.ds(wid * TILE_EDGES, TILE_EDGES)], sraw)

    def start_didx(j, b):
        pltpu.async_copy(
            edge_hbm.at[pl.ds(E + wid * TILE_EDGES + j * CH, CH)],
            didx[b], isem[b])

    def wait_didx(b):
        pltpu.make_async_copy(edge_hbm.at[pl.ds(0, CH)],
                              didx[b], isem[b]).wait()

    def start_gather(j, b):
        pltpu.async_copy(h_hbm.at[sraw.at[pl.ds(j * CH, CH)]],
                         rows[b], gsem[b])

    def wait_gather(b):
        pltpu.make_async_copy(h_hbm.at[sraw.at[pl.ds(0, CH)]],
                              rows[b], gsem[b]).wait()

    def start_scatter(b):
        pltpu.async_copy(rows[b], acc.at[didx[b]], ssem[b], add=True)

    def wait_scatter(b):
        pltpu.make_async_copy(rows[b], acc.at[didx[b]], ssem[b]).wait()

    start_didx(0, 0)
    start_didx(1, 1)
    start_gather(0, 0)
    start_gather(1, 1)

    _fill_rows(zeros_v, ZR, D, 0.0)
    _zero_span(zeros_v, acc, sid * ROWS_MAIN, ROWS_MAIN, ZR)

    @pl.when(sid == NS - 1)
    def _():
        _zero_span(zeros_v, acc, ROWS_MAIN * NS, ROWS_TAIL, ZR)

    plsc.subcore_barrier()

    def step(j, b, prefetch):
        wait_gather(b)
        wait_didx(b)
        start_scatter(b)
        if prefetch:
            wait_scatter(b)
            start_didx(j + 2, b)
            start_gather(j + 2, b)

    def body(g, _):
        step(2 * g, 0, prefetch=True)
        step(2 * g + 1, 1, prefetch=True)
        return 0

    lax.fori_loop(0, (TILE_CHUNKS - 2) // 2, body, 0)
    step(TILE_CHUNKS - 2, 0, prefetch=False)
    step(TILE_CHUNKS - 1, 1, prefetch=False)
    wait_scatter(0)
    wait_scatter(1)

    @pl.when(wid < XTRA)
    def _():
        xb = (NW * TILE_CHUNKS + wid) * CH
        pltpu.sync_copy(edge_hbm.at[pl.ds(xb, CH)], sidx_x)
        pltpu.sync_copy(edge_hbm.at[pl.ds(E + xb, CH)], didx_x)
        pltpu.async_copy(h_hbm.at[sidx_x], r0_v, g0).wait()
        pltpu.async_copy(r0_v, acc.at[didx_x], s0, add=True)
        pltpu.make_async_copy(r0_v, acc.at[didx_x], s0).wait()

    plsc.subcore_barrier()
    r0 = sid * ROWS_MAIN
    pltpu.sync_copy(acc.at[pl.ds(r0, ROWS_MAIN)],
                    part_out.at[cid, pl.ds(r0, ROWS_MAIN)])

    @pl.when(sid == NS - 1)
    def _():
        t0 = ROWS_MAIN * NS
        pltpu.sync_copy(acc.at[pl.ds(t0, ROWS_TAIL)],
                        part_out.at[cid, pl.ds(t0, ROWS_TAIL)])


def _norm_column(cnt0, cnt1):
    deg = cnt0 + cnt1
    norm = jax.lax.rsqrt(jnp.maximum(deg, 1.0))
    norm_t = jnp.swapaxes(norm, 0, 1)
    cols = [norm_t[:, s:s + 1] for s in range(NP2 // CH)]
    return jnp.concatenate(cols, axis=0)[:N]


def _scale_body(node_ref, cnt_ref, h_ref):
    h_ref[...] = node_ref[...] * _norm_column(cnt_ref[0], cnt_ref[1])


_scale_kernel = pl.pallas_call(
    _scale_body,
    out_shape=jax.ShapeDtypeStruct((N, D), jnp.float32),
)


def _combine_body(part_ref, cnt_ref, out_ref):
    agg = part_ref[0] + part_ref[1]
    out_ref[...] = agg * _norm_column(cnt_ref[0], cnt_ref[1])


_combine_kernel = pl.pallas_call(
    _combine_body,
    out_shape=jax.ShapeDtypeStruct((N, D), jnp.float32),
)


def kernel(node_f, edge_index):
    edge_flat = edge_index.astype(jnp.int32).reshape(-1)
    cnt = _degree_kernel(edge_flat).reshape(NC, 2, NP2 // CH, CH)
    h = _scale_kernel(node_f, cnt[:, 0])
    partials = _aggregate_kernel(h, edge_flat)
    return _combine_kernel(partials, cnt[:, 1])


# speedup vs baseline: 1.2232x; 1.0204x over previous
import functools

import jax
import jax.numpy as jnp
from jax import lax
from jax.experimental import pallas as pl
from jax.experimental.pallas import tpu as pltpu
from jax.experimental.pallas import tpu_sc as plsc

N = 10000
D = 128
E = 320000

NC, NS, L = 2, 16, 16
NW = NC * NS
CH = 128
NCHUNK = E // CH
TILE_CHUNKS = NCHUNK // NW
XTRA = NCHUNK - TILE_CHUNKS * NW
TILE_EDGES = TILE_CHUNKS * CH

NP2 = 10240
CROWS = NP2 // NS

ROWS_MAIN = 624
ROWS_TAIL = N - ROWS_MAIN * NS

ZR = 16


def _fill_rows(ref, nrows, ncols, value):
    per_row = ncols // L

    def body(k, _):
        i = k // per_row
        j = k % per_row
        ref[i, pl.ds(j * L, L)] = jnp.full((L,), value, ref.dtype)
        return 0

    lax.fori_loop(0, nrows * per_row, body, 0)


def _fill_1d(ref, n, value):
    for j in range(n // L):
        ref[pl.ds(j * L, L)] = jnp.full((L,), value, ref.dtype)


def _zero_span(zeros_v, dst, start, nrows, zrows):
    full, rem = nrows // zrows, nrows % zrows
    for k in range(full):
        pltpu.sync_copy(zeros_v, dst.at[pl.ds(start + k * zrows, zrows)])
    if rem:
        pltpu.sync_copy(zeros_v.at[pl.ds(0, rem)],
                        dst.at[pl.ds(start + full * zrows, rem)])


_sc_mesh = plsc.VectorSubcoreMesh(core_axis_name="c", subcore_axis_name="s")


@functools.partial(
    pl.kernel,
    out_type=jax.ShapeDtypeStruct((NC * 2 * NP2,), jnp.float32),
    mesh=_sc_mesh,
    scratch_types=(
        pltpu.VMEM_SHARED((NP2,), jnp.float32),
        pltpu.VMEM_SHARED((NP2,), jnp.float32),
        pltpu.VMEM((TILE_EDGES,), jnp.int32),
        pltpu.VMEM((TILE_EDGES,), jnp.int32),
        pltpu.VMEM((TILE_CHUNKS, CH), jnp.int32),
        pltpu.VMEM((TILE_CHUNKS, CH), jnp.int32),
        pltpu.VMEM((CH,), jnp.int32),
        pltpu.VMEM((CH,), jnp.int32),
        pltpu.VMEM((CH,), jnp.float32),
        pltpu.VMEM((CROWS,), jnp.float32),
        pltpu.SemaphoreType.DMA,
    ),
)
def _degree_kernel(edge_hbm, cnt_out,
                   cnt_src, cnt_dst, sraw, draw, sidx, didx,
                   sidx_x, didx_x, ones_v, zeros_v, sem):
    cid = lax.axis_index("c")
    sid = lax.axis_index("s")
    wid = sid * NC + cid

    pltpu.sync_copy(edge_hbm.at[pl.ds(wid * TILE_EDGES, TILE_EDGES)], sraw)
    pltpu.sync_copy(edge_hbm.at[pl.ds(E + wid * TILE_EDGES, TILE_EDGES)], draw)

    @pl.when(wid < XTRA)
    def _():
        xb = (NW * TILE_CHUNKS + wid) * CH
        pltpu.sync_copy(edge_hbm.at[pl.ds(xb, CH)], sidx_x)
        pltpu.sync_copy(edge_hbm.at[pl.ds(E + xb, CH)], didx_x)

    def reshape_body(k, _):
        r = k // (CH // L)
        c = k % (CH // L)
        v = k * L
        sidx[r, pl.ds(c * L, L)] = sraw[pl.ds(v, L)]
        didx[r, pl.ds(c * L, L)] = draw[pl.ds(v, L)]
        return 0

    lax.fori_loop(0, TILE_EDGES // L, reshape_body, 0)

    _fill_1d(ones_v, CH, 1.0)
    _fill_1d(zeros_v, CROWS, 0.0)
    _zero_span(zeros_v, cnt_src, sid * CROWS, CROWS, CROWS)
    _zero_span(zeros_v, cnt_dst, sid * CROWS, CROWS, CROWS)
    plsc.subcore_barrier()

    def issue(i, _):
        pltpu.async_copy(ones_v, cnt_src.at[sidx.at[i]], sem, add=True)
        pltpu.async_copy(ones_v, cnt_dst.at[didx.at[i]], sem, add=True)
        return 0

    lax.fori_loop(0, TILE_CHUNKS, issue, 0)

    @pl.when(wid < XTRA)
    def _():
        pltpu.async_copy(ones_v, cnt_src.at[sidx_x], sem, add=True)
        pltpu.async_copy(ones_v, cnt_dst.at[didx_x], sem, add=True)

    def drain(i, _):
        pltpu.make_async_copy(ones_v, cnt_src.at[sidx.at[i]], sem).wait()
        pltpu.make_async_copy(ones_v, cnt_dst.at[didx.at[i]], sem).wait()
        return 0

    lax.fori_loop(0, TILE_CHUNKS, drain, 0)

    @pl.when(wid < XTRA)
    def _():
        pltpu.make_async_copy(ones_v, cnt_src.at[sidx_x], sem).wait()
        pltpu.make_async_copy(ones_v, cnt_dst.at[didx_x], sem).wait()

    plsc.subcore_barrier()

    def readout(cnt, out_base):
        r0 = sid * CROWS
        pltpu.sync_copy(cnt.at[pl.ds(r0, CROWS)], zeros_v)
        pltpu.sync_copy(zeros_v, cnt_out.at[pl.ds(out_base + r0, CROWS)])

    readout(cnt_src, cid * 2 * NP2)
    readout(cnt_dst, cid * 2 * NP2 + NP2)


NSLOT = 3
NIDX = 6
IDX_AHEAD = 4


@functools.partial(
    pl.kernel,
    out_type=jax.ShapeDtypeStruct((NC, N, D), jnp.float32),
    mesh=_sc_mesh,
    scratch_types=(
        pltpu.VMEM_SHARED((N, D), jnp.float32),
        tuple(pltpu.VMEM((CH,), jnp.int32) for _ in range(NIDX)),
        tuple(pltpu.VMEM((CH,), jnp.int32) for _ in range(NIDX)),
        tuple(pltpu.VMEM((CH, D), jnp.float32) for _ in range(NSLOT)),
        tuple(pltpu.SemaphoreType.DMA for _ in range(NIDX)),
        tuple(pltpu.SemaphoreType.DMA for _ in range(NIDX)),
        tuple(pltpu.SemaphoreType.DMA for _ in range(NSLOT)),
        tuple(pltpu.SemaphoreType.DMA for _ in range(NSLOT)),
    ),
)
def _aggregate_kernel(h_hbm, edge_hbm, part_out,
                      acc, sidx, didx, rows, issem, idsem, gsem, ssem):
    cid = lax.axis_index("c")
    sid = lax.axis_index("s")
    wid = sid * NC + cid

    def start_idx(j, q):
        base = wid * TILE_EDGES + j * CH
        pltpu.async_copy(edge_hbm.at[pl.ds(base, CH)], sidx[q], issem[q])
        pltpu.async_copy(edge_hbm.at[pl.ds(E + base, CH)], didx[q], idsem[q])

    def wait_idx(q):
        pltpu.make_async_copy(edge_hbm.at[pl.ds(0, CH)],
                              sidx[q], issem[q]).wait()
        pltpu.make_async_copy(edge_hbm.at[pl.ds(0, CH)],
                              didx[q], idsem[q]).wait()

    def start_gather(q, b):
        pltpu.async_copy(h_hbm.at[sidx[q]], rows[b], gsem[b])

    def wait_gather(b):
        pltpu.make_async_copy(h_hbm.at[sidx[0]], rows[b], gsem[b]).wait()

    def start_scatter(q, b):
        pltpu.async_copy(rows[b], acc.at[didx[q]], ssem[b], add=True)

    def wait_scatter(b):
        pltpu.make_async_copy(rows[b], acc.at[didx[0]], ssem[b]).wait()

    for j in range(IDX_AHEAD):
        start_idx(j, j)
    wait_idx(0)
    wait_idx(1)
    start_gather(0, 0)
    start_gather(1, 1)

    _fill_rows(rows[2], CH, D, 0.0)
    _zero_span(rows[2], acc, sid * ROWS_MAIN, ROWS_MAIN, CH)

    @pl.when(sid == NS - 1)
    def _():
        _zero_span(rows[2], acc, ROWS_MAIN * NS, ROWS_TAIL, CH)

    plsc.subcore_barrier()

    def step(j, b, q, drain_prev, prefetch):
        wait_gather(b)
        start_scatter(q, b)
        p = (b + 2) % NSLOT
        if drain_prev:
            wait_scatter(p)
        if prefetch:
            @pl.when(j + IDX_AHEAD < TILE_CHUNKS)
            def _():
                start_idx(j + IDX_AHEAD, (q + IDX_AHEAD) % NIDX)

            q2 = (q + 2) % NIDX
            wait_idx(q2)
            start_gather(q2, p)

    step(0, 0, 0, drain_prev=False, prefetch=True)

    def body(g, _):
        j0 = 1 + 6 * g
        for k in range(6):
            jk = k + 1
            step(j0 + k, jk % NSLOT, jk % NIDX,
                 drain_prev=True, prefetch=True)
        return 0

    lax.fori_loop(0, (TILE_CHUNKS - 6) // 6, body, 0)
    for j in range(TILE_CHUNKS - 5, TILE_CHUNKS):
        step(j, j % NSLOT, j % NIDX, drain_prev=True,
             prefetch=(j + 2 < TILE_CHUNKS))
    wait_scatter((TILE_CHUNKS - 1) % NSLOT)

    @pl.when(wid < XTRA)
    def _():
        xb = (NW * TILE_CHUNKS + wid) * CH
        pltpu.sync_copy(edge_hbm.at[pl.ds(xb, CH)], sidx[0])
        pltpu.sync_copy(edge_hbm.at[pl.ds(E + xb, CH)], didx[0])
        pltpu.async_copy(h_hbm.at[sidx[0]], rows[0], gsem[0]).wait()
        pltpu.async_copy(rows[0], acc.at[didx[0]], ssem[0], add=True)
        pltpu.make_async_copy(rows[0], acc.at[didx[0]], ssem[0]).wait()

    plsc.subcore_barrier()
    r0 = sid * ROWS_MAIN
    pltpu.sync_copy(acc.at[pl.ds(r0, ROWS_MAIN)],
                    part_out.at[cid, pl.ds(r0, ROWS_MAIN)])

    @pl.when(sid == NS - 1)
    def _():
        t0 = ROWS_MAIN * NS
        pltpu.sync_copy(acc.at[pl.ds(t0, ROWS_TAIL)],
                        part_out.at[cid, pl.ds(t0, ROWS_TAIL)])


def _norm_column(cnt0, cnt1):
    deg = cnt0 + cnt1
    norm = jax.lax.rsqrt(jnp.maximum(deg, 1.0))
    norm_t = jnp.swapaxes(norm, 0, 1)
    cols = [norm_t[:, s:s + 1] for s in range(NP2 // CH)]
    return jnp.concatenate(cols, axis=0)[:N]


def _scale_body(node_ref, cnt_ref, h_ref):
    h_ref[...] = node_ref[...] * _norm_column(cnt_ref[0], cnt_ref[1])


_scale_kernel = pl.pallas_call(
    _scale_body,
    out_shape=jax.ShapeDtypeStruct((N, D), jnp.float32),
)


def _combine_body(part_ref, cnt_ref, out_ref):
    agg = part_ref[0] + part_ref[1]
    out_ref[...] = agg * _norm_column(cnt_ref[0], cnt_ref[1])


_combine_kernel = pl.pallas_call(
    _combine_body,
    out_shape=jax.ShapeDtypeStruct((N, D), jnp.float32),
)


def kernel(node_f, edge_index):
    edge_flat = edge_index.astype(jnp.int32).reshape(-1)
    cnt = _degree_kernel(edge_flat).reshape(NC, 2, NP2 // CH, CH)
    h = _scale_kernel(node_f, cnt[:, 0])
    partials = _aggregate_kernel(h, edge_flat)
    return _combine_kernel(partials, cnt[:, 1])
